# Initial kernel scaffold; baseline (speedup 1.0000x reference)
import functools

import jax
import jax.numpy as jnp
from jax import lax
from jax.experimental import pallas as pl
from jax.experimental.pallas import tpu as pltpu
from jax.experimental.pallas import tpu_sc as plsc

NC = 2
NS = 16
LANES = 16
CHUNK = 80


def _lin2(x, wrelT, wrootT, b):
    n, d = x.shape
    ho = wrelT.shape[1]
    R = 1000

    def body(x_ref, a_ref, c_ref, b_ref, y_ref, r_ref):
        xb = x_ref[...]
        y_ref[...] = jnp.dot(xb, a_ref[...], preferred_element_type=jnp.float32)
        r_ref[...] = jnp.dot(xb, c_ref[...], preferred_element_type=jnp.float32) + b_ref[...]

    return pl.pallas_call(
        body,
        grid=(n // R,),
        in_specs=[
            pl.BlockSpec((R, d), lambda i: (i, 0)),
            pl.BlockSpec((d, ho), lambda i: (0, 0)),
            pl.BlockSpec((d, ho), lambda i: (0, 0)),
            pl.BlockSpec((1, ho), lambda i: (0, 0)),
        ],
        out_specs=[pl.BlockSpec((R, ho), lambda i: (i, 0))] * 2,
        out_shape=[jax.ShapeDtypeStruct((n, ho), jnp.float32)] * 2,
    )(x, wrelT, wrootT, b.reshape(1, -1))


def _combine_lin(p, wrelT, wrootT, b):
    _, n, w = p.shape
    ho = wrelT.shape[1]
    R = 1000

    def body(p_ref, a_ref, c_ref, b_ref, y_ref, r_ref):
        h = jnp.maximum(p_ref[0, :, :] + p_ref[1, :, :], 0.0)
        y_ref[...] = jnp.dot(h, a_ref[...], preferred_element_type=jnp.float32)
        r_ref[...] = jnp.dot(h, c_ref[...], preferred_element_type=jnp.float32) + b_ref[...]

    return pl.pallas_call(
        body,
        grid=(n // R,),
        in_specs=[
            pl.BlockSpec((2, R, w), lambda i: (0, i, 0)),
            pl.BlockSpec((w, ho), lambda i: (0, 0)),
            pl.BlockSpec((w, ho), lambda i: (0, 0)),
            pl.BlockSpec((1, ho), lambda i: (0, 0)),
        ],
        out_specs=[pl.BlockSpec((R, ho), lambda i: (i, 0))] * 2,
        out_shape=[jax.ShapeDtypeStruct((n, ho), jnp.float32)] * 2,
    )(p, wrelT, wrootT, b.reshape(1, -1))


def _final_sum(p):
    _, n, w = p.shape
    R = 1000

    def body(p_ref, o_ref):
        o_ref[...] = p_ref[0, :, :] + p_ref[1, :, :]

    return pl.pallas_call(
        body,
        grid=(n // R,),
        in_specs=[pl.BlockSpec((2, R, w), lambda i: (0, i, 0))],
        out_specs=pl.BlockSpec((R, w), lambda i: (i, 0)),
        out_shape=jax.ShapeDtypeStruct((n, w), jnp.float32),
    )(p)


def _sc_segsum(y, src3, dst3, w3, r):
    n, w = y.shape
    _, _, g, chunk = src3.shape
    rpt = n // NS
    nj = w // LANES
    z = jnp.zeros((rpt, w), jnp.float32)
    mesh = plsc.VectorSubcoreMesh(core_axis_name="c", subcore_axis_name="s")

    @functools.partial(
        pl.kernel,
        out_type=jax.ShapeDtypeStruct((NC, n, w), jnp.float32),
        mesh=mesh,
        scratch_types=[
            pltpu.VMEM((g, chunk), jnp.int32),
            pltpu.VMEM((g, chunk), jnp.int32),
            pltpu.VMEM((g, chunk), jnp.float32),
            pltpu.VMEM((chunk, w), jnp.float32),
            pltpu.VMEM_SHARED((n, w), jnp.float32),
            pltpu.SemaphoreType.DMA,
        ],
    )
    def seg(y_hbm, src_hbm, dst_hbm, w_hbm, r_hbm, z_hbm, out_hbm,
            src_v, dst_v, w_v, rows_v, acc, sem):
        c = lax.axis_index("c")
        s = lax.axis_index("s")
        sl = pl.ds(s * rpt, rpt)

        @pl.when(c == 0)
        def _():
            pltpu.sync_copy(r_hbm.at[sl], acc.at[sl])

        @pl.when(c != 0)
        def _():
            pltpu.sync_copy(z_hbm, acc.at[sl])

        pltpu.sync_copy(src_hbm.at[c, s], src_v)
        pltpu.sync_copy(dst_hbm.at[c, s], dst_v)
        pltpu.sync_copy(w_hbm.at[c, s], w_v)
        plsc.subcore_barrier()

        def chunk_body(gi, carry):
            pltpu.async_copy(y_hbm.at[src_v.at[gi]], rows_v, sem).wait()
            for e in range(chunk):
                ws = w_v[gi, e]
                for j in range(nj):
                    d = pl.ds(j * LANES, LANES)
                    rows_v[e, d] = rows_v[e, d] * ws
            pltpu.sync_copy(rows_v, acc.at[dst_v.at[gi]], add=True)
            return carry

        lax.fori_loop(0, g, chunk_body, 0)
        plsc.subcore_barrier()
        pltpu.sync_copy(acc.at[sl], out_hbm.at[c].at[sl])

    return seg(y, src3, dst3, w3, r, z)


def kernel(data, x, edge_index, edge_attr,
           W1_rel, b1_rel, W1_root,
           W2_rel, b2_rel, W2_root,
           W3_rel, b3_rel, W3_root):
    e = edge_index.shape[1]
    per_tile = e // (NC * NS)
    g = per_tile // CHUNK
    assert e == NC * NS * g * CHUNK

    src3 = edge_index[0].astype(jnp.int32).reshape(NC, NS, g, CHUNK)
    dst3 = edge_index[1].astype(jnp.int32).reshape(NC, NS, g, CHUNK)
    w3 = edge_attr.astype(jnp.float32).reshape(NC, NS, g, CHUNK)

    y1, r1 = _lin2(x, W1_rel.T, W1_root.T, b1_rel)
    p1 = _sc_segsum(y1, src3, dst3, w3, r1)
    y2, r2 = _combine_lin(p1, W2_rel.T, W2_root.T, b2_rel)
    p2 = _sc_segsum(y2, src3, dst3, w3, r2)
    y3, r3 = _combine_lin(p2, W3_rel.T, W3_root.T, b3_rel)
    p3 = _sc_segsum(y3, src3, dst3, w3, r3)
    return _final_sum(p3)


# speedup vs baseline: 8.2423x; 8.2423x over previous
import functools

import jax
import jax.numpy as jnp
from jax import lax
from jax.experimental import pallas as pl
from jax.experimental.pallas import tpu as pltpu
from jax.experimental.pallas import tpu_sc as plsc

NC = 2
NS = 16
LANES = 16
CHUNK = 80


def _pad_rows(n):
    q = NS * 8
    return ((n + q - 1) // q) * q


def _lin2(x, wrelT, wrootT, b, npad):
    n, d = x.shape
    ho = wrelT.shape[1]
    R = 1000

    def body(x_ref, a_ref, c_ref, b_ref, y_ref, r_ref):
        xb = x_ref[...]
        y_ref[...] = jnp.dot(xb, a_ref[...], preferred_element_type=jnp.float32)
        r_ref[...] = jnp.dot(xb, c_ref[...], preferred_element_type=jnp.float32) + b_ref[...]

    return pl.pallas_call(
        body,
        grid=(n // R,),
        in_specs=[
            pl.BlockSpec((R, d), lambda i: (i, 0)),
            pl.BlockSpec((d, ho), lambda i: (0, 0)),
            pl.BlockSpec((d, ho), lambda i: (0, 0)),
            pl.BlockSpec((1, ho), lambda i: (0, 0)),
        ],
        out_specs=[pl.BlockSpec((R, ho), lambda i: (i, 0))] * 2,
        out_shape=[jax.ShapeDtypeStruct((npad, ho), jnp.float32)] * 2,
    )(x, wrelT, wrootT, b.reshape(1, -1))


def _combine_lin(p, wrelT, wrootT, b, n, npad):
    _, _, w = p.shape
    ho = wrelT.shape[1]
    R = 1000

    def body(p_ref, a_ref, c_ref, b_ref, y_ref, r_ref):
        h = jnp.maximum(p_ref[0, :, :] + p_ref[1, :, :], 0.0)
        y_ref[...] = jnp.dot(h, a_ref[...], preferred_element_type=jnp.float32)
        r_ref[...] = jnp.dot(h, c_ref[...], preferred_element_type=jnp.float32) + b_ref[...]

    return pl.pallas_call(
        body,
        grid=(n // R,),
        in_specs=[
            pl.BlockSpec((2, R, w), lambda i: (0, i, 0)),
            pl.BlockSpec((w, ho), lambda i: (0, 0)),
            pl.BlockSpec((w, ho), lambda i: (0, 0)),
            pl.BlockSpec((1, ho), lambda i: (0, 0)),
        ],
        out_specs=[pl.BlockSpec((R, ho), lambda i: (i, 0))] * 2,
        out_shape=[jax.ShapeDtypeStruct((npad, ho), jnp.float32)] * 2,
    )(p, wrelT, wrootT, b.reshape(1, -1))


def _final_sum(p, n):
    _, _, w = p.shape
    R = 1000

    def body(p_ref, o_ref):
        o_ref[...] = p_ref[0, :, :] + p_ref[1, :, :]

    return pl.pallas_call(
        body,
        grid=(n // R,),
        in_specs=[pl.BlockSpec((2, R, w), lambda i: (0, i, 0))],
        out_specs=pl.BlockSpec((R, w), lambda i: (i, 0)),
        out_shape=jax.ShapeDtypeStruct((n, w), jnp.float32),
    )(p)


def _sc_segsum(y, src3, dst3, w3, r):
    npad, w = y.shape
    _, _, g, chunk = src3.shape
    rpt = npad // NS
    nj = w // LANES
    z = jnp.zeros((rpt, w), jnp.float32)
    mesh = plsc.VectorSubcoreMesh(core_axis_name="c", subcore_axis_name="s")

    @functools.partial(
        pl.kernel,
        out_type=jax.ShapeDtypeStruct((NC, npad, w), jnp.float32),
        mesh=mesh,
        scratch_types=[
            pltpu.VMEM((g, chunk), jnp.int32),
            pltpu.VMEM((g, chunk), jnp.int32),
            pltpu.VMEM((g, chunk), jnp.float32),
            pltpu.VMEM((chunk, w), jnp.float32),
            pltpu.VMEM_SHARED((npad, w), jnp.float32),
            pltpu.SemaphoreType.DMA,
        ],
        compiler_params=pltpu.CompilerParams(use_tc_tiling_on_sc=False),
    )
    def seg(y_hbm, src_hbm, dst_hbm, w_hbm, r_hbm, z_hbm, out_hbm,
            src_v, dst_v, w_v, rows_v, acc, sem):
        c = lax.axis_index("c")
        s = lax.axis_index("s")
        sl = pl.ds(s * rpt, rpt)

        @pl.when(c == 0)
        def _():
            pltpu.sync_copy(r_hbm.at[sl], acc.at[sl])

        @pl.when(c != 0)
        def _():
            pltpu.sync_copy(z_hbm, acc.at[sl])

        pltpu.sync_copy(src_hbm.at[c, s], src_v)
        pltpu.sync_copy(dst_hbm.at[c, s], dst_v)
        pltpu.sync_copy(w_hbm.at[c, s], w_v)
        plsc.subcore_barrier()

        def chunk_body(gi, carry):
            pltpu.async_copy(y_hbm.at[src_v.at[gi]], rows_v, sem).wait()
            for t in range(chunk // LANES):
                wvec = w_v[gi, pl.ds(t * LANES, LANES)]
                for k in range(LANES):
                    e = t * LANES + k
                    ws = wvec[k]
                    for j in range(nj):
                        d = pl.ds(j * LANES, LANES)
                        rows_v[e, d] = rows_v[e, d] * ws
            pltpu.sync_copy(rows_v, acc.at[dst_v.at[gi]], add=True)
            return carry

        lax.fori_loop(0, g, chunk_body, 0)
        plsc.subcore_barrier()
        pltpu.sync_copy(acc.at[sl], out_hbm.at[c].at[sl])

    return seg(y, src3, dst3, w3, r, z)


def kernel(data, x, edge_index, edge_attr,
           W1_rel, b1_rel, W1_root,
           W2_rel, b2_rel, W2_root,
           W3_rel, b3_rel, W3_root):
    n = x.shape[0]
    npad = _pad_rows(n)
    e = edge_index.shape[1]
    per_tile = e // (NC * NS)
    g = per_tile // CHUNK
    assert e == NC * NS * g * CHUNK

    src3 = edge_index[0].astype(jnp.int32).reshape(NC, NS, g, CHUNK)
    dst3 = edge_index[1].astype(jnp.int32).reshape(NC, NS, g, CHUNK)
    w3 = edge_attr.astype(jnp.float32).reshape(NC, NS, g, CHUNK)

    y1, r1 = _lin2(x, W1_rel.T, W1_root.T, b1_rel, npad)
    p1 = _sc_segsum(y1, src3, dst3, w3, r1)
    y2, r2 = _combine_lin(p1, W2_rel.T, W2_root.T, b2_rel, n, npad)
    p2 = _sc_segsum(y2, src3, dst3, w3, r2)
    y3, r3 = _combine_lin(p2, W3_rel.T, W3_root.T, b3_rel, n, npad)
    p3 = _sc_segsum(y3, src3, dst3, w3, r3)
    return _final_sum(p3, n)


# speedup vs baseline: 13.0369x; 1.5817x over previous
import functools

import jax
import jax.numpy as jnp
from jax import lax
from jax.experimental import pallas as pl
from jax.experimental.pallas import tpu as pltpu
from jax.experimental.pallas import tpu_sc as plsc

NC = 2
NS = 16
LANES = 16
CHUNK = 100


def _pad_rows(n):
    q = NS * 8
    return ((n + q - 1) // q) * q


def _lin2(x, wrelT, wrootT, b, npad):
    n, d = x.shape
    ho = wrelT.shape[1]
    R = 1000

    def body(x_ref, a_ref, c_ref, b_ref, y_ref, r_ref):
        xb = x_ref[...]
        y_ref[...] = jnp.dot(xb, a_ref[...], preferred_element_type=jnp.float32)
        r_ref[...] = jnp.dot(xb, c_ref[...], preferred_element_type=jnp.float32) + b_ref[...]

    return pl.pallas_call(
        body,
        grid=(n // R,),
        in_specs=[
            pl.BlockSpec((R, d), lambda i: (i, 0)),
            pl.BlockSpec((d, ho), lambda i: (0, 0)),
            pl.BlockSpec((d, ho), lambda i: (0, 0)),
            pl.BlockSpec((1, ho), lambda i: (0, 0)),
        ],
        out_specs=[pl.BlockSpec((R, ho), lambda i: (i, 0))] * 2,
        out_shape=[jax.ShapeDtypeStruct((npad, ho), jnp.float32)] * 2,
    )(x, wrelT, wrootT, b.reshape(1, -1))


def _combine_lin(p, wrelT, wrootT, b, n, npad):
    _, _, w = p.shape
    ho = wrelT.shape[1]
    R = 1000

    def body(p_ref, a_ref, c_ref, b_ref, y_ref, r_ref):
        h = jnp.maximum(p_ref[0, :, :] + p_ref[1, :, :], 0.0)
        y_ref[...] = jnp.dot(h, a_ref[...], preferred_element_type=jnp.float32)
        r_ref[...] = jnp.dot(h, c_ref[...], preferred_element_type=jnp.float32) + b_ref[...]

    return pl.pallas_call(
        body,
        grid=(n // R,),
        in_specs=[
            pl.BlockSpec((2, R, w), lambda i: (0, i, 0)),
            pl.BlockSpec((w, ho), lambda i: (0, 0)),
            pl.BlockSpec((w, ho), lambda i: (0, 0)),
            pl.BlockSpec((1, ho), lambda i: (0, 0)),
        ],
        out_specs=[pl.BlockSpec((R, ho), lambda i: (i, 0))] * 2,
        out_shape=[jax.ShapeDtypeStruct((npad, ho), jnp.float32)] * 2,
    )(p, wrelT, wrootT, b.reshape(1, -1))


def _final_sum(p, n):
    _, _, w = p.shape
    R = 1000

    def body(p_ref, o_ref):
        o_ref[...] = p_ref[0, :, :] + p_ref[1, :, :]

    return pl.pallas_call(
        body,
        grid=(n // R,),
        in_specs=[pl.BlockSpec((2, R, w), lambda i: (0, i, 0))],
        out_specs=pl.BlockSpec((R, w), lambda i: (i, 0)),
        out_shape=jax.ShapeDtypeStruct((n, w), jnp.float32),
    )(p)


def _sc_segsum(y, src3, dst3, w3, r):
    npad, w = y.shape
    _, _, g, chunk = src3.shape
    rpt = npad // NS
    nj = w // LANES
    z = jnp.zeros((rpt, w), jnp.float32)
    mesh = plsc.VectorSubcoreMesh(core_axis_name="c", subcore_axis_name="s")

    @functools.partial(
        pl.kernel,
        out_type=jax.ShapeDtypeStruct((NC, npad, w), jnp.float32),
        mesh=mesh,
        scratch_types=[
            pltpu.VMEM((g, chunk), jnp.int32),
            pltpu.VMEM((g, chunk), jnp.int32),
            pltpu.VMEM((g, chunk), jnp.float32),
            pltpu.VMEM((chunk, w), jnp.float32),
            pltpu.VMEM((chunk, w), jnp.float32),
            pltpu.VMEM_SHARED((npad, w), jnp.float32),
            pltpu.SemaphoreType.DMA,
            pltpu.SemaphoreType.DMA,
        ],
        compiler_params=pltpu.CompilerParams(use_tc_tiling_on_sc=False),
    )
    def seg(y_hbm, src_hbm, dst_hbm, w_hbm, r_hbm, z_hbm, out_hbm,
            src_v, dst_v, w_v, rows0_v, rows1_v, acc, sem0, sem1):
        c = lax.axis_index("c")
        s = lax.axis_index("s")
        sl = pl.ds(s * rpt, rpt)

        @pl.when(c == 0)
        def _():
            pltpu.sync_copy(r_hbm.at[sl], acc.at[sl])

        @pl.when(c != 0)
        def _():
            pltpu.sync_copy(z_hbm, acc.at[sl])

        pltpu.sync_copy(src_hbm.at[c, s], src_v)
        pltpu.sync_copy(dst_hbm.at[c, s], dst_v)
        pltpu.sync_copy(w_hbm.at[c, s], w_v)
        plsc.subcore_barrier()

        def scale(gi, rows_v):
            for t in range(chunk // LANES):
                wvec = w_v[gi, pl.ds(t * LANES, LANES)]
                for k in range(LANES):
                    e = t * LANES + k
                    ws = wvec[k]
                    for j in range(nj):
                        d = pl.ds(j * LANES, LANES)
                        rows_v[e, d] = rows_v[e, d] * ws

        pltpu.async_copy(y_hbm.at[src_v.at[0]], rows0_v, sem0)

        def pair_body(gp, carry):
            g0 = 2 * gp
            g1 = g0 + 1
            gather1 = pltpu.async_copy(y_hbm.at[src_v.at[g1]], rows1_v, sem1)
            pltpu.make_async_copy(y_hbm.at[src_v.at[g0]], rows0_v, sem0).wait()
            scale(g0, rows0_v)
            pltpu.sync_copy(rows0_v, acc.at[dst_v.at[g0]], add=True)

            @pl.when(g1 + 1 < g)
            def _():
                pltpu.async_copy(y_hbm.at[src_v.at[g1 + 1]], rows0_v, sem0)

            gather1.wait()
            scale(g1, rows1_v)
            pltpu.sync_copy(rows1_v, acc.at[dst_v.at[g1]], add=True)
            return carry

        lax.fori_loop(0, g // 2, pair_body, 0)
        plsc.subcore_barrier()
        pltpu.sync_copy(acc.at[sl], out_hbm.at[c].at[sl])

    return seg(y, src3, dst3, w3, r, z)


def kernel(data, x, edge_index, edge_attr,
           W1_rel, b1_rel, W1_root,
           W2_rel, b2_rel, W2_root,
           W3_rel, b3_rel, W3_root):
    n = x.shape[0]
    npad = _pad_rows(n)
    e = edge_index.shape[1]
    per_tile = e // (NC * NS)
    g = per_tile // CHUNK
    assert e == NC * NS * g * CHUNK

    src3 = edge_index[0].astype(jnp.int32).reshape(NC, NS, g, CHUNK)
    dst3 = edge_index[1].astype(jnp.int32).reshape(NC, NS, g, CHUNK)
    w3 = edge_attr.astype(jnp.float32).reshape(NC, NS, g, CHUNK)

    y1, r1 = _lin2(x, W1_rel.T, W1_root.T, b1_rel, npad)
    p1 = _sc_segsum(y1, src3, dst3, w3, r1)
    y2, r2 = _combine_lin(p1, W2_rel.T, W2_root.T, b2_rel, n, npad)
    p2 = _sc_segsum(y2, src3, dst3, w3, r2)
    y3, r3 = _combine_lin(p2, W3_rel.T, W3_root.T, b3_rel, n, npad)
    p3 = _sc_segsum(y3, src3, dst3, w3, r3)
    return _final_sum(p3, n)
